# baseline (speedup 1.0000x reference)
import functools

import jax
import jax.numpy as jnp
from jax import lax
from jax.experimental import pallas as pl
from jax.experimental.pallas import tpu as pltpu
from jax.experimental.pallas import tpu_sc as plsc

B = 16384
F = 26
FEAT_NUM = 100000
K = 16
FEATURE_LENGTH = F * FEAT_NUM
RPB = 8
NBLK = FEATURE_LENGTH // RPB

NC = 2
NS = 16
NW = NC * NS

CPB = 8
NCOL = 20313
LASTBASE = NCOL - CPB
NSLOT = (NCOL + CPB - 1) // CPB
NPAIR = (NSLOT + 2 * NW - 1) // (2 * NW)

RPW = B // NW
CH = 16
NCH = RPW // CH
CF = CH * F

_mesh = plsc.VectorSubcoreMesh(core_axis_name="c", subcore_axis_name="s")
_params = pltpu.CompilerParams(needs_layout_passes=False)


TCC = 1024
TCGRID = -(-FEATURE_LENGTH // TCC)


def _tc_body(x_ref, o_ref):
    t = x_ref[...].T.reshape(TCC // 8, 8, K)
    o_ref[...] = jnp.concatenate([t[:, s, :] for s in range(8)], axis=1)


_to_blocks = pl.pallas_call(
    _tc_body,
    grid=(TCGRID,),
    in_specs=[pl.BlockSpec((K, TCC), lambda i: (0, i))],
    out_specs=pl.BlockSpec((TCC // 8, 128), lambda i: (i, 0)),
    out_shape=jax.ShapeDtypeStruct((NBLK, 128), jnp.float32),
)


@functools.partial(
    pl.kernel,
    out_type=jax.ShapeDtypeStruct((B,), jnp.float32),
    mesh=_mesh,
    compiler_params=_params,
    scratch_types=[
        pltpu.VMEM((RPW * F,), jnp.int32),
        pltpu.VMEM((RPW * F,), jnp.int32),
        pltpu.VMEM((CF, 128), jnp.float32),
        pltpu.VMEM((CF,), jnp.float32),
        pltpu.VMEM((RPW,), jnp.float32),
        pltpu.SemaphoreType.DMA,
        pltpu.SemaphoreType.DMA,
    ],
)
def _fm_sc(idx_hbm, w_hbm, v_hbm, out_hbm, idx_v, blk_v, vrows, wrows, out_v,
           semv, semw):
    wid = lax.axis_index("s") * NC + lax.axis_index("c")
    base = wid * RPW

    pltpu.sync_copy(idx_hbm.at[pl.ds(base * F, RPW * F)], idx_v)

    @pl.loop(0, RPW * F // 16)
    def _blk(i):
        sl = pl.ds(i * 16, 16)
        blk_v[sl] = lax.shift_right_logical(idx_v[sl], 3)

    iota = lax.iota(jnp.int32, 16)
    zero = jnp.zeros((16,), jnp.float32)

    @pl.loop(0, NCH)
    def _chunk(ch):
        cpv = pltpu.async_copy(
            v_hbm.at[blk_v.at[pl.ds(ch * CF, CF)]], vrows, semv)
        cpw = pltpu.async_copy(
            w_hbm.at[idx_v.at[pl.ds(ch * CF, CF)]], wrows, semw)
        cpv.wait()
        cpw.wait()

        fidx = [iota * F + f for f in range(F)]

        wacc = zero
        sub16 = []
        for f in range(F):
            wacc = wacc + plsc.load_gather(wrows, [fidx[f]])
            g = plsc.load_gather(idx_v, [ch * CF + fidx[f]])
            sub16.append(lax.shift_left(jnp.bitwise_and(g, 7), 4))

        t2 = zero
        tot = zero
        for k in range(K):
            acc = zero
            for f in range(F):
                v = plsc.load_gather(vrows, [fidx[f], sub16[f] + k])
                acc = acc + v
                t2 = t2 + v * v
            tot = tot + acc * acc

        res = wacc + 0.5 * (tot - t2)
        out_v[pl.ds(ch * CH, 16)] = res

    pltpu.sync_copy(out_v, out_hbm.at[pl.ds(base, RPW)])


def kernel(inputs, w0, w, V):
    offsets = (jnp.arange(F, dtype=jnp.int32) * FEAT_NUM)[None, :]
    idx = (inputs.astype(jnp.int32) + offsets).reshape(-1)
    vblk = _to_blocks(V.T)
    out = _fm_sc(idx, w.reshape(-1), vblk)
    return out[:, None] + w0


# speedup vs baseline: 2.0737x; 2.0737x over previous
import functools

import jax
import jax.numpy as jnp
from jax import lax
from jax.experimental import pallas as pl
from jax.experimental.pallas import tpu as pltpu
from jax.experimental.pallas import tpu_sc as plsc

B = 16384
F = 26
FEAT_NUM = 100000
K = 16
FEATURE_LENGTH = F * FEAT_NUM
RPB = 8
NBLK = FEATURE_LENGTH // RPB

NC = 2
NS = 16
NW = NC * NS

CPB = 8
NCOL = 20313
LASTBASE = NCOL - CPB
NSLOT = (NCOL + CPB - 1) // CPB
NPAIR = (NSLOT + 2 * NW - 1) // (2 * NW)

RPW = B // NW
CH = 16
NCH = RPW // CH
CF = CH * F

_mesh = plsc.VectorSubcoreMesh(core_axis_name="c", subcore_axis_name="s")
_params = pltpu.CompilerParams(needs_layout_passes=False)


NBLK_PAD = NCOL * 16


@functools.partial(
    pl.kernel,
    out_type=jax.ShapeDtypeStruct((NBLK_PAD, 128), jnp.float32),
    mesh=_mesh,
    compiler_params=_params,
    scratch_types=[
        pltpu.VMEM((2, K, CPB * 128), jnp.float32),
        pltpu.VMEM((2, CPB * 16, 128), jnp.float32),
        pltpu.SemaphoreType.DMA,
        pltpu.SemaphoreType.DMA,
        pltpu.SemaphoreType.DMA,
        pltpu.SemaphoreType.DMA,
    ],
)
def _to_blocks(vt_hbm, out_hbm, vin, vout, si0, si1, so0, so1):
    wid = lax.axis_index("s") * NC + lax.axis_index("c")
    iota = lax.iota(jnp.int32, 16)
    zeros = jnp.zeros((16,), jnp.int32)

    def colbase(slot):
        return pl.multiple_of(jnp.minimum(slot * CPB, LASTBASE) * 128, 1024)

    def fire_in(slot, buf, sem):
        return pltpu.async_copy(
            vt_hbm.at[:, pl.ds(colbase(slot), CPB * 128)], vin.at[buf], sem)

    def fire_out(slot, buf, sem):
        return pltpu.async_copy(
            vout.at[buf],
            out_hbm.at[pl.ds(pl.multiple_of(colbase(slot) // 8, 128),
                             CPB * 16), :],
            sem)

    def wait_in(buf, sem):
        pltpu.make_async_copy(
            vt_hbm.at[:, pl.ds(0, CPB * 128)], vin.at[buf], sem).wait()

    def wait_out(buf, sem):
        pltpu.make_async_copy(
            vout.at[buf], out_hbm.at[pl.ds(0, CPB * 16), :], sem).wait()

    s16iotas = [iota + s * 16 for s in range(8)]

    def transpose(buf):
        src = vin.at[buf]
        dst = vout.at[buf]

        @plsc.parallel_loop(0, CPB, unroll=2)
        def _cc(cc):
            def loads(u2):
                vecs = []
                for u in (2 * u2, 2 * u2 + 1):
                    for s in range(8):
                        col = cc * 128 + u * 8 + s
                        vecs.append(plsc.load_gather(src, [iota, zeros + col]))
                return vecs

            def stores(u2, vecs):
                i = 0
                for u in (2 * u2, 2 * u2 + 1):
                    usplat = zeros + (cc * 16 + u)
                    for s in range(8):
                        plsc.store_scatter(dst, [usplat, s16iotas[s]], vecs[i])
                        i += 1

            vecs = loads(0)
            for u2 in range(7):
                nxt = loads(u2 + 1)
                stores(u2, vecs)
                vecs = nxt
            stores(7, vecs)

    fire_in(wid, 0, si0)

    @pl.loop(0, NPAIR)
    def _pair(j):
        s0 = wid + (2 * j) * NW
        s1 = s0 + NW
        fire_in(s1, 1, si1)
        wait_in(0, si0)

        @pl.when(j > 0)
        def _():
            wait_out(0, so0)
        transpose(0)
        fire_out(s0, 0, so0)

        @pl.when(j < NPAIR - 1)
        def _():
            fire_in(s1 + NW, 0, si0)
        wait_in(1, si1)

        @pl.when(j > 0)
        def _():
            wait_out(1, so1)
        transpose(1)
        fire_out(s1, 1, so1)

    wait_out(0, so0)
    wait_out(1, so1)


@functools.partial(
    pl.kernel,
    out_type=jax.ShapeDtypeStruct((B,), jnp.float32),
    mesh=_mesh,
    compiler_params=_params,
    scratch_types=[
        pltpu.VMEM((RPW * F,), jnp.int32),
        pltpu.VMEM((RPW * F,), jnp.int32),
        pltpu.VMEM((CF, 128), jnp.float32),
        pltpu.VMEM((CF,), jnp.float32),
        pltpu.VMEM((RPW,), jnp.float32),
        pltpu.SemaphoreType.DMA,
        pltpu.SemaphoreType.DMA,
    ],
)
def _fm_sc(idx_hbm, w_hbm, v_hbm, out_hbm, idx_v, blk_v, vrows, wrows, out_v,
           semv, semw):
    wid = lax.axis_index("s") * NC + lax.axis_index("c")
    base = wid * RPW

    pltpu.sync_copy(idx_hbm.at[pl.ds(base * F, RPW * F)], idx_v)

    @pl.loop(0, RPW * F // 16)
    def _blk(i):
        sl = pl.ds(i * 16, 16)
        blk_v[sl] = lax.shift_right_logical(idx_v[sl], 3)

    iota = lax.iota(jnp.int32, 16)
    zero = jnp.zeros((16,), jnp.float32)

    @pl.loop(0, NCH)
    def _chunk(ch):
        cpv = pltpu.async_copy(
            v_hbm.at[blk_v.at[pl.ds(ch * CF, CF)]], vrows, semv)
        cpw = pltpu.async_copy(
            w_hbm.at[idx_v.at[pl.ds(ch * CF, CF)]], wrows, semw)
        cpv.wait()
        cpw.wait()

        fidx = [iota * F + f for f in range(F)]

        wacc = zero
        sub16 = []
        for f in range(F):
            wacc = wacc + plsc.load_gather(wrows, [fidx[f]])
            g = plsc.load_gather(idx_v, [ch * CF + fidx[f]])
            sub16.append(lax.shift_left(jnp.bitwise_and(g, 7), 4))

        t2 = zero
        tot = zero
        for k in range(K):
            acc = zero
            for f in range(F):
                v = plsc.load_gather(vrows, [fidx[f], sub16[f] + k])
                acc = acc + v
                t2 = t2 + v * v
            tot = tot + acc * acc

        res = wacc + 0.5 * (tot - t2)
        out_v[pl.ds(ch * CH, 16)] = res

    pltpu.sync_copy(out_v, out_hbm.at[pl.ds(base, RPW)])


def kernel(inputs, w0, w, V):
    offsets = (jnp.arange(F, dtype=jnp.int32) * FEAT_NUM)[None, :]
    idx = (inputs.astype(jnp.int32) + offsets).reshape(-1)
    vblk = _to_blocks(V.T)
    out = _fm_sc(idx, w.reshape(-1), vblk)
    return out[:, None] + w0


# speedup vs baseline: 2.5686x; 1.2386x over previous
import functools

import jax
import jax.numpy as jnp
from jax import lax
from jax.experimental import pallas as pl
from jax.experimental.pallas import tpu as pltpu
from jax.experimental.pallas import tpu_sc as plsc

B = 16384
F = 26
FEAT_NUM = 100000
K = 16
FEATURE_LENGTH = F * FEAT_NUM
RPB = 8
NBLK = FEATURE_LENGTH // RPB

NC = 2
NS = 16
NW = NC * NS

CPB = 8
NCOL = 20313
LASTBASE = NCOL - CPB
NSLOT = (NCOL + CPB - 1) // CPB
NPAIR = (NSLOT + 2 * NW - 1) // (2 * NW)

RPW = B // NW
CH = 16
NCH = RPW // CH
CF = CH * F

_mesh = plsc.VectorSubcoreMesh(core_axis_name="c", subcore_axis_name="s")
_params = pltpu.CompilerParams(needs_layout_passes=False)


NBLK_PAD = NCOL * 16


@functools.partial(
    pl.kernel,
    out_type=jax.ShapeDtypeStruct((NBLK_PAD, 128), jnp.float32),
    mesh=_mesh,
    compiler_params=_params,
    scratch_types=[
        pltpu.VMEM((4, K, CPB * 128), jnp.float32),
        pltpu.VMEM((2, CPB * 16, 128), jnp.float32),
        pltpu.SemaphoreType.DMA,
        pltpu.SemaphoreType.DMA,
        pltpu.SemaphoreType.DMA,
        pltpu.SemaphoreType.DMA,
        pltpu.SemaphoreType.DMA,
        pltpu.SemaphoreType.DMA,
    ],
)
def _to_blocks(vt_hbm, out_hbm, vin, vout, si0, si1, si2, si3, so0, so1):
    wid = lax.axis_index("s") * NC + lax.axis_index("c")
    iota = lax.iota(jnp.int32, 16)
    zeros = jnp.zeros((16,), jnp.int32)

    def colbase(slot):
        return pl.multiple_of(jnp.minimum(slot * CPB, LASTBASE) * 128, 1024)

    def fire_in(slot, buf, sem):
        return pltpu.async_copy(
            vt_hbm.at[:, pl.ds(colbase(slot), CPB * 128)], vin.at[buf], sem)

    def fire_out(slot, buf, sem):
        return pltpu.async_copy(
            vout.at[buf],
            out_hbm.at[pl.ds(pl.multiple_of(colbase(slot) // 8, 128),
                             CPB * 16), :],
            sem)

    def wait_in(buf, sem):
        pltpu.make_async_copy(
            vt_hbm.at[:, pl.ds(0, CPB * 128)], vin.at[buf], sem).wait()

    def wait_out(buf, sem):
        pltpu.make_async_copy(
            vout.at[buf], out_hbm.at[pl.ds(0, CPB * 16), :], sem).wait()

    s16iotas = [iota + s * 16 for s in range(8)]

    def transpose(inb, outb):
        src = vin.at[inb]
        dst = vout.at[outb]

        @plsc.parallel_loop(0, CPB, unroll=2)
        def _cc(cc):
            for u2 in range(8):
                vecs = []
                for u in (2 * u2, 2 * u2 + 1):
                    for s in range(8):
                        col = cc * 128 + u * 8 + s
                        vecs.append(plsc.load_gather(src, [iota, zeros + col]))
                i = 0
                for u in (2 * u2, 2 * u2 + 1):
                    usplat = zeros + (cc * 16 + u)
                    for s in range(8):
                        plsc.store_scatter(dst, [usplat, s16iotas[s]], vecs[i])
                        i += 1

    sis = (si0, si1, si2, si3)
    sos = (so0, so1)
    NSL = 2 * NPAIR
    for m0 in range(3):
        fire_in(wid + m0 * NW, m0, sis[m0])

    @pl.loop(0, NPAIR // 2)
    def _quad(j):
        for t in range(4):
            m = 4 * j + t
            slot = wid + m * NW

            @pl.when(m + 3 < NSL)
            def _():
                fire_in(wid + (m + 3) * NW, (t + 3) % 4, sis[(t + 3) % 4])
            wait_in(t, sis[t])

            @pl.when(m >= 2)
            def _():
                wait_out(t % 2, sos[t % 2])
            transpose(t, t % 2)
            fire_out(slot, t % 2, sos[t % 2])

    wait_out(0, so0)
    wait_out(1, so1)


@functools.partial(
    pl.kernel,
    out_type=jax.ShapeDtypeStruct((B,), jnp.float32),
    mesh=_mesh,
    compiler_params=_params,
    scratch_types=[
        pltpu.VMEM((RPW * F,), jnp.int32),
        pltpu.VMEM((RPW * F,), jnp.int32),
        pltpu.VMEM((CF, 128), jnp.float32),
        pltpu.VMEM((CF,), jnp.float32),
        pltpu.VMEM((RPW,), jnp.float32),
        pltpu.SemaphoreType.DMA,
        pltpu.SemaphoreType.DMA,
    ],
)
def _fm_sc(idx_hbm, w_hbm, v_hbm, out_hbm, idx_v, blk_v, vrows, wrows, out_v,
           semv, semw):
    wid = lax.axis_index("s") * NC + lax.axis_index("c")
    base = wid * RPW

    pltpu.sync_copy(idx_hbm.at[pl.ds(base * F, RPW * F)], idx_v)

    @pl.loop(0, RPW * F // 16)
    def _blk(i):
        sl = pl.ds(i * 16, 16)
        blk_v[sl] = lax.shift_right_logical(idx_v[sl], 3)

    iota = lax.iota(jnp.int32, 16)
    zero = jnp.zeros((16,), jnp.float32)

    @pl.loop(0, NCH)
    def _chunk(ch):
        cpv = pltpu.async_copy(
            v_hbm.at[blk_v.at[pl.ds(ch * CF, CF)]], vrows, semv)
        cpw = pltpu.async_copy(
            w_hbm.at[idx_v.at[pl.ds(ch * CF, CF)]], wrows, semw)
        cpv.wait()
        cpw.wait()

        fidx = [iota * F + f for f in range(F)]

        wacc = zero
        sub16 = []
        for f in range(F):
            wacc = wacc + plsc.load_gather(wrows, [fidx[f]])
            g = plsc.load_gather(idx_v, [ch * CF + fidx[f]])
            sub16.append(lax.shift_left(jnp.bitwise_and(g, 7), 4))

        t2 = zero
        tot = zero
        for k in range(K):
            acc = zero
            for f in range(F):
                v = plsc.load_gather(vrows, [fidx[f], sub16[f] + k])
                acc = acc + v
                t2 = t2 + v * v
            tot = tot + acc * acc

        res = wacc + 0.5 * (tot - t2)
        out_v[pl.ds(ch * CH, 16)] = res

    pltpu.sync_copy(out_v, out_hbm.at[pl.ds(base, RPW)])


def kernel(inputs, w0, w, V):
    offsets = (jnp.arange(F, dtype=jnp.int32) * FEAT_NUM)[None, :]
    idx = (inputs.astype(jnp.int32) + offsets).reshape(-1)
    vblk = _to_blocks(V.T)
    out = _fm_sc(idx, w.reshape(-1), vblk)
    return out[:, None] + w0


# speedup vs baseline: 2.8971x; 1.1279x over previous
import functools

import jax
import jax.numpy as jnp
from jax import lax
from jax.experimental import pallas as pl
from jax.experimental.pallas import tpu as pltpu
from jax.experimental.pallas import tpu_sc as plsc

B = 16384
F = 26
FEAT_NUM = 100000
K = 16
FEATURE_LENGTH = F * FEAT_NUM
RPB = 8
NBLK = FEATURE_LENGTH // RPB

NC = 2
NS = 16
NW = NC * NS

CPB = 8
NCOL = 20313
LASTBASE = NCOL - CPB
NSLOT = (NCOL + CPB - 1) // CPB
NPAIR = (NSLOT + 2 * NW - 1) // (2 * NW)
NBLK_PAD = NCOL * 16

RPW = B // NW
CH = 16
NCH = RPW // CH
CF = CH * F

_mesh = plsc.VectorSubcoreMesh(core_axis_name="c", subcore_axis_name="s")
_params = pltpu.CompilerParams(needs_layout_passes=False)


@functools.partial(
    pl.kernel,
    out_type=jax.ShapeDtypeStruct((NBLK_PAD, 128), jnp.float32),
    mesh=_mesh,
    compiler_params=_params,
    scratch_types=[
        pltpu.VMEM((2, K, CPB * 128), jnp.float32),
        pltpu.VMEM((2, CPB * 16, 128), jnp.float32),
        pltpu.SemaphoreType.DMA,
        pltpu.SemaphoreType.DMA,
        pltpu.SemaphoreType.DMA,
        pltpu.SemaphoreType.DMA,
    ],
)
def _to_blocks(vt_hbm, out_hbm, vin, vout, si0, si1, so0, so1):
    wid = lax.axis_index("s") * NC + lax.axis_index("c")
    iota = lax.iota(jnp.int32, 16)
    zeros = jnp.zeros((16,), jnp.int32)

    def colbase(slot):
        return pl.multiple_of(jnp.minimum(slot * CPB, LASTBASE) * 128, 1024)

    def fire_in(slot, buf, sem):
        return pltpu.async_copy(
            vt_hbm.at[:, pl.ds(colbase(slot), CPB * 128)], vin.at[buf], sem)

    def fire_out(slot, buf, sem):
        return pltpu.async_copy(
            vout.at[buf],
            out_hbm.at[pl.ds(pl.multiple_of(colbase(slot) // 8, 128),
                             CPB * 16), :],
            sem)

    def wait_in(buf, sem):
        pltpu.make_async_copy(
            vt_hbm.at[:, pl.ds(0, CPB * 128)], vin.at[buf], sem).wait()

    def wait_out(buf, sem):
        pltpu.make_async_copy(
            vout.at[buf], out_hbm.at[pl.ds(0, CPB * 16), :], sem).wait()

    s16iotas = [iota + s * 16 for s in range(8)]

    def transpose(buf):
        @plsc.parallel_loop(0, CPB, unroll=2)
        def _cc(cc):
            src = vin.at[buf]
            dst = vout.at[buf]
            for u2 in range(8):
                vecs = []
                for u in (2 * u2, 2 * u2 + 1):
                    for s in range(8):
                        col = cc * 128 + u * 8 + s
                        vecs.append(plsc.load_gather(src, [iota, zeros + col]))
                i = 0
                for u in (2 * u2, 2 * u2 + 1):
                    usplat = zeros + (cc * 16 + u)
                    for s in range(8):
                        plsc.store_scatter(dst, [usplat, s16iotas[s]], vecs[i])
                        i += 1

    fire_in(wid, 0, si0)

    @pl.loop(0, NPAIR)
    def _pair(j):
        s0 = wid + (2 * j) * NW
        s1 = s0 + NW
        fire_in(s1, 1, si1)
        wait_in(0, si0)

        @pl.when(j > 0)
        def _():
            wait_out(0, so0)
        transpose(0)
        fire_out(s0, 0, so0)

        @pl.when(j < NPAIR - 1)
        def _():
            fire_in(s1 + NW, 0, si0)
        wait_in(1, si1)

        @pl.when(j > 0)
        def _():
            wait_out(1, so1)
        transpose(1)
        fire_out(s1, 1, so1)

    wait_out(0, so0)
    wait_out(1, so1)


@functools.partial(
    pl.kernel,
    out_type=jax.ShapeDtypeStruct((B,), jnp.float32),
    mesh=_mesh,
    compiler_params=_params,
    scratch_types=[
        pltpu.VMEM((CF,), jnp.int32),
        pltpu.VMEM((CF,), jnp.int32),
        pltpu.VMEM((CF,), jnp.int32),
        pltpu.VMEM((CF,), jnp.int32),
        pltpu.VMEM((CF, 128), jnp.float32),
        pltpu.VMEM((CF, 128), jnp.float32),
        pltpu.VMEM((CF,), jnp.float32),
        pltpu.VMEM((CF,), jnp.float32),
        pltpu.VMEM((RPW,), jnp.float32),
        pltpu.SemaphoreType.DMA,
        pltpu.SemaphoreType.DMA,
        pltpu.SemaphoreType.DMA,
        pltpu.SemaphoreType.DMA,
    ],
)
def _fm_sc(idx_hbm, w_hbm, v_hbm, out_hbm, idx0, idx1, blk0, blk1, vrows0,
           vrows1, wrows0, wrows1, out_v, sv0, sv1, sw0, sw1):
    wid = lax.axis_index("s") * NC + lax.axis_index("c")
    base = wid * RPW

    iota = lax.iota(jnp.int32, 16)
    zero = jnp.zeros((16,), jnp.float32)
    svs = (sv0, sv1)
    sws = (sw0, sw1)
    vbufs = (vrows0, vrows1)
    wbufs = (wrows0, wrows1)
    bbufs = (blk0, blk1)
    ibufs = (idx0, idx1)

    def fire(ch, buf):
        pltpu.sync_copy(
            idx_hbm.at[pl.ds((base + ch * CH) * F, CF)], ibufs[buf])

        @pl.loop(0, CF // 16)
        def _blk(i):
            sl = pl.ds(i * 16, 16)
            bbufs[buf][sl] = lax.shift_right_logical(ibufs[buf][sl], 3)

        pltpu.async_copy(v_hbm.at[bbufs[buf]], vbufs[buf], svs[buf])
        pltpu.async_copy(w_hbm.at[ibufs[buf]], wbufs[buf], sws[buf])

    def wait_bufs(buf):
        pltpu.make_async_copy(
            v_hbm.at[bbufs[buf]], vbufs[buf], svs[buf]).wait()
        pltpu.make_async_copy(
            w_hbm.at[ibufs[buf]], wbufs[buf], sws[buf]).wait()

    def compute(ch, buf):
        wacc = zero
        t2 = zero
        accs = [zero] * K
        for f in range(F):
            fi = iota * F + f
            wacc = wacc + plsc.load_gather(wbufs[buf], [fi])
            g = plsc.load_gather(ibufs[buf], [fi])
            s16 = lax.shift_left(jnp.bitwise_and(g, 7), 4)
            for k in range(K):
                v = plsc.load_gather(vbufs[buf], [fi, s16 + k])
                accs[k] = accs[k] + v
                t2 = t2 + v * v

        tot = zero
        for k in range(K):
            tot = tot + accs[k] * accs[k]

        res = wacc + 0.5 * (tot - t2)
        out_v[pl.ds(ch * CH, 16)] = res

    fire(0, 0)

    @pl.loop(0, NCH // 2)
    def _chunk(c2):
        ch0 = c2 * 2
        fire(ch0 + 1, 1)
        wait_bufs(0)
        compute(ch0, 0)

        @pl.when(c2 < NCH // 2 - 1)
        def _():
            fire(ch0 + 2, 0)
        wait_bufs(1)
        compute(ch0 + 1, 1)

    pltpu.sync_copy(out_v, out_hbm.at[pl.ds(base, RPW)])


def kernel(inputs, w0, w, V):
    offsets = (jnp.arange(F, dtype=jnp.int32) * FEAT_NUM)[None, :]
    idx = (inputs.astype(jnp.int32) + offsets).reshape(-1)
    vblk = _to_blocks(V.T)
    out = _fm_sc(idx, w.reshape(-1), vblk)
    return out[:, None] + w0
